# baseline (speedup 1.0000x reference)
import functools
import math

import jax
import jax.numpy as jnp
from jax.experimental import pallas as pl
from jax.experimental.pallas import tpu as pltpu

_BS, _N = 4, 2048
_K = 32
_NEG = -1e30
_HI = jax.lax.Precision.HIGHEST
_PAR = pltpu.CompilerParams(dimension_semantics=("parallel",))


def _cat_bias(bs):
    return jnp.concatenate(bs, axis=0)[None, :]


def _block_diag(ws):
    rows = sum(w.shape[0] for w in ws)
    cols = sum(w.shape[1] for w in ws)
    out = jnp.zeros((rows, cols), jnp.float32)
    r = c = 0
    for w in ws:
        out = out.at[r:r + w.shape[0], c:c + w.shape[1]].set(w)
        r += w.shape[0]
        c += w.shape[1]
    return out


def _proj_body(x_ref, b1_ref, c1_ref, b2_ref, c2_ref, wc1_ref, bc1_ref,
               wc2_ref, bc2_ref, o_ref):
    x = x_ref[0]
    h = jnp.maximum(jnp.dot(x, b1_ref[...], precision=_HI) + c1_ref[...], 0.0)
    h = jnp.dot(h, b2_ref[...], precision=_HI) + c2_ref[...]
    t = jnp.maximum(jnp.dot(h, wc1_ref[...], precision=_HI) + bc1_ref[...], 0.0)
    cst = jnp.dot(t, wc2_ref[...], precision=_HI) + bc2_ref[...]
    for j in range(6):
        o_ref[0, j] = h[:, 8 * j:8 * j + 8]
    o_ref[0, 6] = cst


def _run_proj(x18, pj):
    b1 = _block_diag([pj[k][0]['W'] for k in ('xyz', 'pmt', 'mad', 'dim', 'nor', 'loc')])
    c1 = _cat_bias([pj[k][0]['b'] for k in ('xyz', 'pmt', 'mad', 'dim', 'nor', 'loc')])
    b2 = _block_diag([pj[k][1]['W'] for k in ('xyz', 'pmt', 'mad', 'dim', 'nor', 'loc')])
    c2 = _cat_bias([pj[k][1]['b'] for k in ('xyz', 'pmt', 'mad', 'dim', 'nor', 'loc')])
    wc1, bc1 = pj['cst'][0]['W'], pj['cst'][0]['b'][None, :]
    wc2, bc2 = pj['cst'][1]['W'], pj['cst'][1]['b'][None, :]
    full = lambda s: pl.BlockSpec(s, lambda b: (0,) * len(s))
    return pl.pallas_call(
        _proj_body,
        grid=(_BS,),
        in_specs=[
            pl.BlockSpec((1, _N, 18), lambda b: (b, 0, 0)),
            full(b1.shape), full(c1.shape), full(b2.shape), full(c2.shape),
            full(wc1.shape), full(bc1.shape), full(wc2.shape), full(bc2.shape),
        ],
        out_specs=pl.BlockSpec((1, 7, _N, 8), lambda b: (b, 0, 0, 0)),
        out_shape=jax.ShapeDtypeStruct((_BS, 7, _N, 8), jnp.float32),
        compiler_params=_PAR,
    )(x18, b1, c1, b2, c2, wc1, bc1, wc2, bc2)


def _fps_body(xt_ref, xr_ref, o_ref, *, n, m, c):
    xt = xt_ref[...]
    sqx = jnp.sum(xt * xt, axis=1)
    lane = jax.lax.broadcasted_iota(jnp.int32, (_BS, n), 1)
    dist0 = jnp.full((_BS, n), 1e10, jnp.float32)
    last0 = jnp.zeros((_BS, 1), jnp.int32)
    o_ref[0, :] = jnp.zeros((_BS,), jnp.int32)

    def step(t, carry):
        dist, last = carry
        curs = [xr_ref[pl.ds(last[b, 0] + b * n, 1), :] for b in range(_BS)]
        cur = jnp.concatenate(curs, axis=0)
        sqc = jnp.sum(cur * cur, axis=1, keepdims=True)
        prod = jnp.sum(xt * cur[:, :, None], axis=1)
        d = sqx - 2.0 * prod + sqc
        dist = jnp.minimum(dist, d)
        mx = jnp.max(dist, axis=1, keepdims=True)
        nxt = jnp.min(jnp.where(dist == mx, lane, n), axis=1, keepdims=True)
        o_ref[t, :] = nxt.T[0]
        return dist, nxt

    jax.lax.fori_loop(1, m, step, (dist0, last0))


def _run_fps(x_t, x_rows, m):
    c, n = x_t.shape[1], x_t.shape[2]
    out = pl.pallas_call(
        functools.partial(_fps_body, n=n, m=m, c=c),
        out_shape=jax.ShapeDtypeStruct((m, _BS), jnp.int32),
    )(x_t, x_rows.reshape(_BS * n, c))
    return out.T


def _stage_body(f_ref, fi_ref, wq_ref, wk_ref, wv_ref, w1_ref, b1_ref,
                w2_ref, b2_ref, fw1_ref, fw2_ref, o_ref, *, n, m, c, cout,
                last_stage):
    f = f_ref[0]
    fi = fi_ref[0]
    lane = jax.lax.broadcasted_iota(jnp.int32, (m, n), 1)
    oh = (lane == fi).astype(jnp.float32)
    fcat = jnp.concatenate([f[j] for j in range(7)], axis=1)
    cen = jnp.dot(oh, fcat, precision=_HI)

    x0 = f[0]
    c0 = cen[:, :c]
    sqc = jnp.sum(c0 * c0, axis=1, keepdims=True)
    sqx = jax.lax.dot_general(
        jnp.ones((1, c), jnp.float32), x0 * x0,
        (((1,), (1,)), ((), ())), precision=_HI)
    cross = jax.lax.dot_general(c0, x0, (((1,), (1,)), ((), ())),
                                precision=_HI)
    d = sqc + sqx - 2.0 * cross

    inf = jnp.float32(jnp.inf)

    def tk(_, dwork):
        mn = jnp.min(dwork, axis=1, keepdims=True)
        idx = jnp.min(jnp.where(dwork == mn, lane, n), axis=1, keepdims=True)
        return jnp.where(lane == idx, inf, dwork)

    dwork = jax.lax.fori_loop(0, _K, tk, d)
    addmask = jnp.where(dwork == inf, 0.0, _NEG)

    scale = 1.0 / math.sqrt(float(c))
    outs = []
    for j in range(7):
        fj = f[j]
        cj = cen[:, j * c:(j + 1) * c]
        q = jnp.dot(cj, wq_ref[j])
        km = jnp.dot(fj, wk_ref[j])
        v = jnp.dot(fj, wv_ref[j])
        logits = jax.lax.dot_general(q, km, (((1,), (1,)), ((), ()))) * scale
        logits = logits + addmask
        mx = jnp.max(logits, axis=1, keepdims=True)
        e = jnp.exp(logits - mx)
        a = e / jnp.sum(e, axis=1, keepdims=True)
        att = jnp.dot(a, v)
        h = jnp.maximum(jnp.dot(att, w1_ref[j]) + b1_ref[j], 0.0)
        h = jnp.dot(h, w2_ref[j]) + b2_ref[j]
        outs.append(h)

    scs = [jnp.dot(jnp.tanh(jnp.dot(h, fw1_ref[...])), fw2_ref[...])
           for h in outs]
    smx = scs[0]
    for s in scs[1:]:
        smx = jnp.maximum(smx, s)
    es = [jnp.exp(s - smx) for s in scs]
    z = es[0]
    for e_ in es[1:]:
        z = z + e_
    fused = es[0] / z * outs[0]
    for j in range(1, 7):
        fused = fused + es[j] / z * outs[j]

    if last_stage:
        o_ref[0] = fused
    else:
        for j in range(6):
            o_ref[0, j] = outs[j]
        o_ref[0, 6] = fused


def _run_stage(feats, fidx, ssa, fea, n, m, c, cout, last_stage):
    wq = jnp.stack([a['Wq'] for a in ssa['attn']])
    wk = jnp.stack([a['Wk'] for a in ssa['attn']])
    wv = jnp.stack([a['Wv'] for a in ssa['attn']])
    w1 = jnp.stack([p[0]['W'] for p in ssa['mlp']])
    b1 = jnp.stack([p[0]['b'][None, :] for p in ssa['mlp']])
    w2 = jnp.stack([p[1]['W'] for p in ssa['mlp']])
    b2 = jnp.stack([p[1]['b'][None, :] for p in ssa['mlp']])
    fw1 = fea['W1']
    fw2 = fea['w2'][:, None]
    fi3 = fidx[:, :, None]
    if last_stage:
        out_shape = jax.ShapeDtypeStruct((_BS, m, cout), jnp.float32)
        out_spec = pl.BlockSpec((1, m, cout), lambda b: (b, 0, 0))
    else:
        out_shape = jax.ShapeDtypeStruct((_BS, 7, m, cout), jnp.float32)
        out_spec = pl.BlockSpec((1, 7, m, cout), lambda b: (b, 0, 0, 0))
    full = lambda s: pl.BlockSpec(s, lambda b: (0,) * len(s))
    return pl.pallas_call(
        functools.partial(_stage_body, n=n, m=m, c=c, cout=cout,
                          last_stage=last_stage),
        grid=(_BS,),
        in_specs=[
            pl.BlockSpec((1, 7, n, c), lambda b: (b, 0, 0, 0)),
            pl.BlockSpec((1, m, 1), lambda b: (b, 0, 0)),
            full(wq.shape), full(wk.shape), full(wv.shape),
            full(w1.shape), full(b1.shape), full(w2.shape), full(b2.shape),
            full(fw1.shape), full(fw2.shape),
        ],
        out_specs=out_spec,
        out_shape=out_shape,
        compiler_params=_PAR,
    )(feats, fi3, wq, wk, wv, w1, b1, w2, b2, fw1, fw2)


def _head_body(x_ref, w1_ref, b1_ref, w2_ref, b2_ref, o_ref):
    g = jnp.max(x_ref[...], axis=1)
    h = jnp.maximum(jnp.dot(g, w1_ref[...], precision=_HI) + b1_ref[...], 0.0)
    o_ref[...] = jnp.dot(h, w2_ref[...], precision=_HI) + b2_ref[...]


def _run_head(fused, head):
    w1, b1 = head[0]['W'], head[0]['b'][None, :]
    w2, b2 = head[1]['W'], head[1]['b'][None, :]
    return pl.pallas_call(
        _head_body,
        out_shape=jax.ShapeDtypeStruct((_BS, w2.shape[1]), jnp.float32),
    )(fused, w1, b1, w2, b2)


def kernel(xyz, pmt, mad, dim, nor, loc, params):
    x18 = jnp.concatenate([xyz, pmt, mad, dim, nor, loc], axis=-1)
    feats = _run_proj(x18, params['proj'])

    cfg = [
        ('ssa1', 'fea1', _N, 1024, 8, 32, False),
        ('ssa2', 'fea2', 1024, 512, 32, 128, False),
        ('ssa3', 'fea3', 512, 256, 128, 256, True),
    ]
    for ssa_k, fea_k, n, m, c, cout, last in cfg:
        x_t = jnp.transpose(feats[:, 0], (0, 2, 1))
        fidx = _run_fps(x_t, feats[:, 0], m)
        feats = _run_stage(feats, fidx, params[ssa_k], params[fea_k],
                           n, m, c, cout, last)

    return _run_head(feats, params['head'])


# speedup vs baseline: 1.0685x; 1.0685x over previous
import functools
import math

import jax
import jax.numpy as jnp
from jax.experimental import pallas as pl
from jax.experimental.pallas import tpu as pltpu

_BS, _N = 4, 2048
_K = 32
_NEG = -1e30
_HI = jax.lax.Precision.HIGHEST
_PAR = pltpu.CompilerParams(dimension_semantics=("parallel",))


def _cat_bias(bs):
    return jnp.concatenate(bs, axis=0)[None, :]


def _block_diag(ws):
    rows = sum(w.shape[0] for w in ws)
    cols = sum(w.shape[1] for w in ws)
    out = jnp.zeros((rows, cols), jnp.float32)
    r = c = 0
    for w in ws:
        out = out.at[r:r + w.shape[0], c:c + w.shape[1]].set(w)
        r += w.shape[0]
        c += w.shape[1]
    return out


def _proj_body(x_ref, b1_ref, c1_ref, b2_ref, c2_ref, wc1_ref, bc1_ref,
               wc2_ref, bc2_ref, o_ref):
    x = x_ref[0]
    h = jnp.maximum(jnp.dot(x, b1_ref[...], precision=_HI) + c1_ref[...], 0.0)
    h = jnp.dot(h, b2_ref[...], precision=_HI) + c2_ref[...]
    t = jnp.maximum(jnp.dot(h, wc1_ref[...], precision=_HI) + bc1_ref[...], 0.0)
    cst = jnp.dot(t, wc2_ref[...], precision=_HI) + bc2_ref[...]
    for j in range(6):
        o_ref[0, j] = h[:, 8 * j:8 * j + 8]
    o_ref[0, 6] = cst


def _run_proj(x18, pj):
    b1 = _block_diag([pj[k][0]['W'] for k in ('xyz', 'pmt', 'mad', 'dim', 'nor', 'loc')])
    c1 = _cat_bias([pj[k][0]['b'] for k in ('xyz', 'pmt', 'mad', 'dim', 'nor', 'loc')])
    b2 = _block_diag([pj[k][1]['W'] for k in ('xyz', 'pmt', 'mad', 'dim', 'nor', 'loc')])
    c2 = _cat_bias([pj[k][1]['b'] for k in ('xyz', 'pmt', 'mad', 'dim', 'nor', 'loc')])
    wc1, bc1 = pj['cst'][0]['W'], pj['cst'][0]['b'][None, :]
    wc2, bc2 = pj['cst'][1]['W'], pj['cst'][1]['b'][None, :]
    full = lambda s: pl.BlockSpec(s, lambda b: (0,) * len(s))
    return pl.pallas_call(
        _proj_body,
        grid=(_BS,),
        in_specs=[
            pl.BlockSpec((1, _N, 18), lambda b: (b, 0, 0)),
            full(b1.shape), full(c1.shape), full(b2.shape), full(c2.shape),
            full(wc1.shape), full(bc1.shape), full(wc2.shape), full(bc2.shape),
        ],
        out_specs=pl.BlockSpec((1, 7, _N, 8), lambda b: (b, 0, 0, 0)),
        out_shape=jax.ShapeDtypeStruct((_BS, 7, _N, 8), jnp.float32),
        compiler_params=_PAR,
    )(x18, b1, c1, b2, c2, wc1, bc1, wc2, bc2)


_UNROLL = 4


def _fps_body(xt_ref, o_ref, *, n, m, c):
    xt = xt_ref[...]
    sqx = jnp.sum(xt * xt, axis=1)
    lane = jax.lax.broadcasted_iota(jnp.int32, (_BS, n), 1)
    miota = jax.lax.broadcasted_iota(jnp.int32, (_BS, m), 1)
    dist0 = jnp.full((_BS, n), 1e10, jnp.float32)
    oh0 = (lane == 0).astype(jnp.float32)
    o0 = jnp.zeros((_BS, m), jnp.int32)

    def one(t, dist, oh, o):
        cur = jnp.sum(xt * oh[:, None, :], axis=2, keepdims=True)
        sqc = jnp.sum(cur * cur, axis=1)
        prod = jnp.sum(xt * cur, axis=1)
        d = sqx - 2.0 * prod + sqc
        dist = jnp.minimum(dist, d)
        mx = jnp.max(dist, axis=1, keepdims=True)
        nxt = jnp.min(jnp.where(dist == mx, lane, n), axis=1, keepdims=True)
        oh = (lane == nxt).astype(jnp.float32)
        o = jnp.where(miota == t, nxt, o)
        return dist, oh, o

    head = (m - 1) % _UNROLL
    dist, oh, o = dist0, oh0, o0
    for t in range(1, 1 + head):
        dist, oh, o = one(t, dist, oh, o)

    def step(i, carry):
        dist, oh, o = carry
        t = 1 + head + i * _UNROLL
        for u in range(_UNROLL):
            dist, oh, o = one(t + u, dist, oh, o)
        return dist, oh, o

    _, _, o = jax.lax.fori_loop(0, (m - 1 - head) // _UNROLL, step,
                                (dist, oh, o))
    o_ref[...] = o


def _run_fps(x_t, m):
    c, n = x_t.shape[1], x_t.shape[2]
    return pl.pallas_call(
        functools.partial(_fps_body, n=n, m=m, c=c),
        out_shape=jax.ShapeDtypeStruct((_BS, m), jnp.int32),
    )(x_t)


def _stage_body(f_ref, fi_ref, wq_ref, wk_ref, wv_ref, w1_ref, b1_ref,
                w2_ref, b2_ref, fw1_ref, fw2_ref, o_ref, *, n, m, c, cout,
                last_stage):
    f = f_ref[0]
    fi = fi_ref[0]
    lane = jax.lax.broadcasted_iota(jnp.int32, (m, n), 1)
    oh = (lane == fi).astype(jnp.float32)
    fcat = jnp.concatenate([f[j] for j in range(7)], axis=1)
    cen = jnp.dot(oh, fcat, precision=_HI)

    x0 = f[0]
    c0 = cen[:, :c]
    sqc = jnp.sum(c0 * c0, axis=1, keepdims=True)
    sqx = jax.lax.dot_general(
        jnp.ones((1, c), jnp.float32), x0 * x0,
        (((1,), (1,)), ((), ())), precision=_HI)
    cross = jax.lax.dot_general(c0, x0, (((1,), (1,)), ((), ())),
                                precision=_HI)
    d = sqc + sqx - 2.0 * cross

    inf = jnp.float32(jnp.inf)

    def tk(_, dwork):
        mn = jnp.min(dwork, axis=1, keepdims=True)
        idx = jnp.min(jnp.where(dwork == mn, lane, n), axis=1, keepdims=True)
        return jnp.where(lane == idx, inf, dwork)

    dwork = jax.lax.fori_loop(0, _K, tk, d)
    addmask = jnp.where(dwork == inf, 0.0, _NEG)

    scale = 1.0 / math.sqrt(float(c))
    outs = []
    for j in range(7):
        fj = f[j]
        cj = cen[:, j * c:(j + 1) * c]
        q = jnp.dot(cj, wq_ref[j])
        km = jnp.dot(fj, wk_ref[j])
        v = jnp.dot(fj, wv_ref[j])
        logits = jax.lax.dot_general(q, km, (((1,), (1,)), ((), ()))) * scale
        logits = logits + addmask
        mx = jnp.max(logits, axis=1, keepdims=True)
        e = jnp.exp(logits - mx)
        a = e / jnp.sum(e, axis=1, keepdims=True)
        att = jnp.dot(a, v)
        h = jnp.maximum(jnp.dot(att, w1_ref[j]) + b1_ref[j], 0.0)
        h = jnp.dot(h, w2_ref[j]) + b2_ref[j]
        outs.append(h)

    scs = [jnp.dot(jnp.tanh(jnp.dot(h, fw1_ref[...])), fw2_ref[...])
           for h in outs]
    smx = scs[0]
    for s in scs[1:]:
        smx = jnp.maximum(smx, s)
    es = [jnp.exp(s - smx) for s in scs]
    z = es[0]
    for e_ in es[1:]:
        z = z + e_
    fused = es[0] / z * outs[0]
    for j in range(1, 7):
        fused = fused + es[j] / z * outs[j]

    if last_stage:
        o_ref[0] = fused
    else:
        for j in range(6):
            o_ref[0, j] = outs[j]
        o_ref[0, 6] = fused


def _run_stage(feats, fidx, ssa, fea, n, m, c, cout, last_stage):
    wq = jnp.stack([a['Wq'] for a in ssa['attn']])
    wk = jnp.stack([a['Wk'] for a in ssa['attn']])
    wv = jnp.stack([a['Wv'] for a in ssa['attn']])
    w1 = jnp.stack([p[0]['W'] for p in ssa['mlp']])
    b1 = jnp.stack([p[0]['b'][None, :] for p in ssa['mlp']])
    w2 = jnp.stack([p[1]['W'] for p in ssa['mlp']])
    b2 = jnp.stack([p[1]['b'][None, :] for p in ssa['mlp']])
    fw1 = fea['W1']
    fw2 = fea['w2'][:, None]
    fi3 = fidx[:, :, None]
    if last_stage:
        out_shape = jax.ShapeDtypeStruct((_BS, m, cout), jnp.float32)
        out_spec = pl.BlockSpec((1, m, cout), lambda b: (b, 0, 0))
    else:
        out_shape = jax.ShapeDtypeStruct((_BS, 7, m, cout), jnp.float32)
        out_spec = pl.BlockSpec((1, 7, m, cout), lambda b: (b, 0, 0, 0))
    full = lambda s: pl.BlockSpec(s, lambda b: (0,) * len(s))
    return pl.pallas_call(
        functools.partial(_stage_body, n=n, m=m, c=c, cout=cout,
                          last_stage=last_stage),
        grid=(_BS,),
        in_specs=[
            pl.BlockSpec((1, 7, n, c), lambda b: (b, 0, 0, 0)),
            pl.BlockSpec((1, m, 1), lambda b: (b, 0, 0)),
            full(wq.shape), full(wk.shape), full(wv.shape),
            full(w1.shape), full(b1.shape), full(w2.shape), full(b2.shape),
            full(fw1.shape), full(fw2.shape),
        ],
        out_specs=out_spec,
        out_shape=out_shape,
        compiler_params=_PAR,
    )(feats, fi3, wq, wk, wv, w1, b1, w2, b2, fw1, fw2)


def _head_body(x_ref, w1_ref, b1_ref, w2_ref, b2_ref, o_ref):
    g = jnp.max(x_ref[...], axis=1)
    h = jnp.maximum(jnp.dot(g, w1_ref[...], precision=_HI) + b1_ref[...], 0.0)
    o_ref[...] = jnp.dot(h, w2_ref[...], precision=_HI) + b2_ref[...]


def _run_head(fused, head):
    w1, b1 = head[0]['W'], head[0]['b'][None, :]
    w2, b2 = head[1]['W'], head[1]['b'][None, :]
    return pl.pallas_call(
        _head_body,
        out_shape=jax.ShapeDtypeStruct((_BS, w2.shape[1]), jnp.float32),
    )(fused, w1, b1, w2, b2)


def kernel(xyz, pmt, mad, dim, nor, loc, params):
    x18 = jnp.concatenate([xyz, pmt, mad, dim, nor, loc], axis=-1)
    feats = _run_proj(x18, params['proj'])

    cfg = [
        ('ssa1', 'fea1', _N, 1024, 8, 32, False),
        ('ssa2', 'fea2', 1024, 512, 32, 128, False),
        ('ssa3', 'fea3', 512, 256, 128, 256, True),
    ]
    for ssa_k, fea_k, n, m, c, cout, last in cfg:
        x_t = jnp.transpose(feats[:, 0], (0, 2, 1))
        fidx = _run_fps(x_t, m)
        feats = _run_stage(feats, fidx, params[ssa_k], params[fea_k],
                           n, m, c, cout, last)

    return _run_head(feats, params['head'])


# speedup vs baseline: 1.0933x; 1.0232x over previous
import functools
import math

import jax
import jax.numpy as jnp
from jax.experimental import pallas as pl
from jax.experimental.pallas import tpu as pltpu

_BS, _N = 4, 2048
_K = 32
_NEG = -1e30
_HI = jax.lax.Precision.HIGHEST
_PAR = pltpu.CompilerParams(dimension_semantics=("parallel",))


def _cat_bias(bs):
    return jnp.concatenate(bs, axis=0)[None, :]


def _block_diag(ws):
    rows = sum(w.shape[0] for w in ws)
    cols = sum(w.shape[1] for w in ws)
    out = jnp.zeros((rows, cols), jnp.float32)
    r = c = 0
    for w in ws:
        out = out.at[r:r + w.shape[0], c:c + w.shape[1]].set(w)
        r += w.shape[0]
        c += w.shape[1]
    return out


def _proj_body(x_ref, b1_ref, c1_ref, b2_ref, c2_ref, wc1_ref, bc1_ref,
               wc2_ref, bc2_ref, o_ref):
    x = x_ref[0]
    h = jnp.maximum(jnp.dot(x, b1_ref[...], precision=_HI) + c1_ref[...], 0.0)
    h = jnp.dot(h, b2_ref[...], precision=_HI) + c2_ref[...]
    t = jnp.maximum(jnp.dot(h, wc1_ref[...], precision=_HI) + bc1_ref[...], 0.0)
    cst = jnp.dot(t, wc2_ref[...], precision=_HI) + bc2_ref[...]
    for j in range(6):
        o_ref[0, j] = h[:, 8 * j:8 * j + 8]
    o_ref[0, 6] = cst


def _run_proj(x18, pj):
    b1 = _block_diag([pj[k][0]['W'] for k in ('xyz', 'pmt', 'mad', 'dim', 'nor', 'loc')])
    c1 = _cat_bias([pj[k][0]['b'] for k in ('xyz', 'pmt', 'mad', 'dim', 'nor', 'loc')])
    b2 = _block_diag([pj[k][1]['W'] for k in ('xyz', 'pmt', 'mad', 'dim', 'nor', 'loc')])
    c2 = _cat_bias([pj[k][1]['b'] for k in ('xyz', 'pmt', 'mad', 'dim', 'nor', 'loc')])
    wc1, bc1 = pj['cst'][0]['W'], pj['cst'][0]['b'][None, :]
    wc2, bc2 = pj['cst'][1]['W'], pj['cst'][1]['b'][None, :]
    full = lambda s: pl.BlockSpec(s, lambda b: (0,) * len(s))
    return pl.pallas_call(
        _proj_body,
        grid=(_BS,),
        in_specs=[
            pl.BlockSpec((1, _N, 18), lambda b: (b, 0, 0)),
            full(b1.shape), full(c1.shape), full(b2.shape), full(c2.shape),
            full(wc1.shape), full(bc1.shape), full(wc2.shape), full(bc2.shape),
        ],
        out_specs=pl.BlockSpec((1, 7, _N, 8), lambda b: (b, 0, 0, 0)),
        out_shape=jax.ShapeDtypeStruct((_BS, 7, _N, 8), jnp.float32),
        compiler_params=_PAR,
    )(x18, b1, c1, b2, c2, wc1, bc1, wc2, bc2)


_UNROLL = 4


def _fps_body(xt_ref, o_ref, *, n, m, c):
    xt = xt_ref[...]
    sqx = jnp.sum(xt * xt, axis=1)
    lane = jax.lax.broadcasted_iota(jnp.int32, (_BS, n), 1)
    miota = jax.lax.broadcasted_iota(jnp.int32, (_BS, m), 1)
    dist0 = jnp.full((_BS, n), 1e10, jnp.float32)
    oh0 = (lane == 0).astype(jnp.float32)
    o0 = jnp.zeros((_BS, m), jnp.int32)

    def one(t, dist, oh, o):
        cur = jnp.sum(xt * oh[:, None, :], axis=2, keepdims=True)
        sqc = jnp.sum(cur * cur, axis=1)
        prod = jnp.sum(xt * cur, axis=1)
        d = sqx - 2.0 * prod + sqc
        dist = jnp.minimum(dist, d)
        mx = jnp.max(dist, axis=1, keepdims=True)
        nxt = jnp.min(jnp.where(dist == mx, lane, n), axis=1, keepdims=True)
        oh = (lane == nxt).astype(jnp.float32)
        o = jnp.where(miota == t, nxt, o)
        return dist, oh, o

    head = (m - 1) % _UNROLL
    dist, oh, o = dist0, oh0, o0
    for t in range(1, 1 + head):
        dist, oh, o = one(t, dist, oh, o)

    def step(i, carry):
        dist, oh, o = carry
        t = 1 + head + i * _UNROLL
        for u in range(_UNROLL):
            dist, oh, o = one(t + u, dist, oh, o)
        return dist, oh, o

    _, _, o = jax.lax.fori_loop(0, (m - 1 - head) // _UNROLL, step,
                                (dist, oh, o))
    o_ref[...] = o


def _run_fps(x_t, m):
    c, n = x_t.shape[1], x_t.shape[2]
    return pl.pallas_call(
        functools.partial(_fps_body, n=n, m=m, c=c),
        out_shape=jax.ShapeDtypeStruct((_BS, m), jnp.int32),
    )(x_t)


def _stage_body(f_ref, fi_ref, wq_ref, wk_ref, wv_ref, w1_ref, b1_ref,
                w2_ref, b2_ref, fw1_ref, fw2_ref, o_ref, *, n, m, c, cout,
                last_stage):
    f = f_ref[0]
    fi = fi_ref[0]
    lane = jax.lax.broadcasted_iota(jnp.int32, (m, n), 1)
    oh = (lane == fi).astype(jnp.float32)
    fcat = jnp.concatenate([f[j] for j in range(7)], axis=1)
    cen = jnp.dot(oh, fcat, precision=_HI)

    x0 = f[0]
    c0 = cen[:, :c]
    sqc = jnp.sum(c0 * c0, axis=1, keepdims=True)
    sqx = jax.lax.dot_general(
        jnp.ones((1, c), jnp.float32), x0 * x0,
        (((1,), (1,)), ((), ())), precision=_HI)
    cross = jax.lax.dot_general(c0, x0, (((1,), (1,)), ((), ())),
                                precision=_HI)
    d = sqc + sqx - 2.0 * cross

    inf = jnp.float32(jnp.inf)

    def tk(_, dwork):
        mn = jnp.min(dwork, axis=1, keepdims=True)
        idx = jnp.min(jnp.where(dwork == mn, lane, n), axis=1, keepdims=True)
        return jnp.where(lane == idx, inf, dwork)

    dwork = jax.lax.fori_loop(0, _K, tk, d)
    addmask = jnp.where(dwork == inf, 0.0, _NEG)

    scale = 1.0 / math.sqrt(float(c))
    outs = []
    for j in range(7):
        fj = f[j]
        cj = cen[:, j * c:(j + 1) * c]
        q = jnp.dot(cj, wq_ref[j]) * scale
        km = jnp.dot(fj, wk_ref[j])
        v = jnp.dot(fj, wv_ref[j])
        logits = jax.lax.dot_general(q, km, (((1,), (1,)), ((), ())))
        e = jnp.exp(logits + addmask)
        z = jnp.sum(e, axis=1, keepdims=True)
        att = jnp.dot(e, v) / z
        h = jnp.maximum(jnp.dot(att, w1_ref[j]) + b1_ref[j], 0.0)
        h = jnp.dot(h, w2_ref[j]) + b2_ref[j]
        outs.append(h)

    scs = [jnp.dot(jnp.tanh(jnp.dot(h, fw1_ref[...])), fw2_ref[...])
           for h in outs]
    smx = scs[0]
    for s in scs[1:]:
        smx = jnp.maximum(smx, s)
    es = [jnp.exp(s - smx) for s in scs]
    z = es[0]
    for e_ in es[1:]:
        z = z + e_
    fused = es[0] / z * outs[0]
    for j in range(1, 7):
        fused = fused + es[j] / z * outs[j]

    if last_stage:
        o_ref[0] = fused
    else:
        for j in range(6):
            o_ref[0, j] = outs[j]
        o_ref[0, 6] = fused


def _run_stage(feats, fidx, ssa, fea, n, m, c, cout, last_stage):
    wq = jnp.stack([a['Wq'] for a in ssa['attn']])
    wk = jnp.stack([a['Wk'] for a in ssa['attn']])
    wv = jnp.stack([a['Wv'] for a in ssa['attn']])
    w1 = jnp.stack([p[0]['W'] for p in ssa['mlp']])
    b1 = jnp.stack([p[0]['b'][None, :] for p in ssa['mlp']])
    w2 = jnp.stack([p[1]['W'] for p in ssa['mlp']])
    b2 = jnp.stack([p[1]['b'][None, :] for p in ssa['mlp']])
    fw1 = fea['W1']
    fw2 = fea['w2'][:, None]
    fi3 = fidx[:, :, None]
    if last_stage:
        out_shape = jax.ShapeDtypeStruct((_BS, m, cout), jnp.float32)
        out_spec = pl.BlockSpec((1, m, cout), lambda b: (b, 0, 0))
    else:
        out_shape = jax.ShapeDtypeStruct((_BS, 7, m, cout), jnp.float32)
        out_spec = pl.BlockSpec((1, 7, m, cout), lambda b: (b, 0, 0, 0))
    full = lambda s: pl.BlockSpec(s, lambda b: (0,) * len(s))
    return pl.pallas_call(
        functools.partial(_stage_body, n=n, m=m, c=c, cout=cout,
                          last_stage=last_stage),
        grid=(_BS,),
        in_specs=[
            pl.BlockSpec((1, 7, n, c), lambda b: (b, 0, 0, 0)),
            pl.BlockSpec((1, m, 1), lambda b: (b, 0, 0)),
            full(wq.shape), full(wk.shape), full(wv.shape),
            full(w1.shape), full(b1.shape), full(w2.shape), full(b2.shape),
            full(fw1.shape), full(fw2.shape),
        ],
        out_specs=out_spec,
        out_shape=out_shape,
        compiler_params=_PAR,
    )(feats, fi3, wq, wk, wv, w1, b1, w2, b2, fw1, fw2)


def _head_body(x_ref, w1_ref, b1_ref, w2_ref, b2_ref, o_ref):
    g = jnp.max(x_ref[...], axis=1)
    h = jnp.maximum(jnp.dot(g, w1_ref[...], precision=_HI) + b1_ref[...], 0.0)
    o_ref[...] = jnp.dot(h, w2_ref[...], precision=_HI) + b2_ref[...]


def _run_head(fused, head):
    w1, b1 = head[0]['W'], head[0]['b'][None, :]
    w2, b2 = head[1]['W'], head[1]['b'][None, :]
    return pl.pallas_call(
        _head_body,
        out_shape=jax.ShapeDtypeStruct((_BS, w2.shape[1]), jnp.float32),
    )(fused, w1, b1, w2, b2)


def kernel(xyz, pmt, mad, dim, nor, loc, params):
    x18 = jnp.concatenate([xyz, pmt, mad, dim, nor, loc], axis=-1)
    feats = _run_proj(x18, params['proj'])

    cfg = [
        ('ssa1', 'fea1', _N, 1024, 8, 32, False),
        ('ssa2', 'fea2', 1024, 512, 32, 128, False),
        ('ssa3', 'fea3', 512, 256, 128, 256, True),
    ]
    for ssa_k, fea_k, n, m, c, cout, last in cfg:
        x_t = jnp.transpose(feats[:, 0], (0, 2, 1))
        fidx = _run_fps(x_t, m)
        feats = _run_stage(feats, fidx, params[ssa_k], params[fea_k],
                           n, m, c, cout, last)

    return _run_head(feats, params['head'])
